# initial kernel scaffold (unmeasured)
import jax
import jax.numpy as jnp
from jax import lax
from jax.experimental import pallas as pl
from jax.experimental.pallas import tpu as pltpu

N_DEV = 32
M_PER = 128
K = 4096
N = 2048


def kernel(x, w_mat, scale_x, scale_w):
    def body(x_ref, w_ref, sx_ref, sw_ref, out_ref, xl_ref,
             send_sems, recv_sems):
        me = lax.axis_index("i")

        rdmas = []
        for s in range(1, N_DEV):
            dst = lax.rem(me + s, N_DEV)
            rdma = pltpu.make_async_remote_copy(
                src_ref=x_ref.at[pl.ds(dst * M_PER, M_PER), :],
                dst_ref=xl_ref.at[:, pl.ds(me * M_PER, M_PER)],
                send_sem=send_sems.at[s],
                recv_sem=recv_sems.at[s],
                device_id=(dst,),
                device_id_type=pl.DeviceIdType.MESH,
            )
            rdma.start()
            rdmas.append(rdma)

        xl_ref[:, pl.ds(me * M_PER, M_PER)] = x_ref[pl.ds(me * M_PER, M_PER), :]

        for rdma in rdmas:
            rdma.wait_recv()
        for rdma in rdmas:
            rdma.wait_send()

        acc = lax.dot_general(
            xl_ref[:, :], w_ref[:, :],
            (((1,), (0,)), ((), ())),
            preferred_element_type=jnp.int32,
        )
        y = acc.astype(jnp.float32) * (sx_ref[0] * sw_ref[0])
        out_ref[:, :] = y * jax.nn.sigmoid(y)

    return pl.pallas_call(
        body,
        out_shape=jax.ShapeDtypeStruct((M_PER, N), jnp.float32),
        in_specs=[
            pl.BlockSpec(memory_space=pltpu.VMEM),
            pl.BlockSpec(memory_space=pltpu.VMEM),
            pl.BlockSpec(memory_space=pltpu.SMEM),
            pl.BlockSpec(memory_space=pltpu.SMEM),
        ],
        out_specs=pl.BlockSpec(memory_space=pltpu.VMEM),
        scratch_shapes=[
            pltpu.VMEM((M_PER, K), jnp.int8),
            pltpu.SemaphoreType.DMA((N_DEV,)),
            pltpu.SemaphoreType.DMA((N_DEV,)),
        ],
        compiler_params=pltpu.CompilerParams(collective_id=0),
    )(x, w_mat, scale_x, scale_w)


# baseline (device time: 28389 ns/iter reference)
import jax
import jax.numpy as jnp
from jax import lax
from jax.experimental import pallas as pl
from jax.experimental.pallas import tpu as pltpu

N_DEV = 32
M_PER = 128
K = 4096
N = 2048


def kernel(x, w_mat, scale_x, scale_w):
    def body(x_ref, w_ref, sx_ref, sw_ref, out_ref, xl_ref,
             send_sems, recv_sems):
        me = lax.axis_index("i")

        rdmas = []
        for s in range(1, N_DEV):
            dst = lax.rem(me + s, N_DEV)
            rdma = pltpu.make_async_remote_copy(
                src_ref=x_ref.at[pl.ds(dst * M_PER, M_PER), :],
                dst_ref=xl_ref.at[:, pl.ds(me * M_PER, M_PER)],
                send_sem=send_sems.at[s],
                recv_sem=recv_sems.at[s],
                device_id=(dst,),
                device_id_type=pl.DeviceIdType.MESH,
            )
            rdma.start()
            rdmas.append(rdma)

        xl_ref[:, pl.ds(me * M_PER, M_PER)] = x_ref[pl.ds(me * M_PER, M_PER), :]

        for rdma in rdmas:
            rdma.wait_recv()
        for rdma in rdmas:
            rdma.wait_send()

        acc = lax.dot_general(
            xl_ref[:, :], w_ref[:, :],
            (((1,), (0,)), ((), ())),
            preferred_element_type=jnp.int32,
        )
        y = acc.astype(jnp.float32) * (sx_ref[0] * sw_ref[0])
        out_ref[:, :] = y * jax.nn.sigmoid(y)

    return pl.pallas_call(
        body,
        out_shape=jax.ShapeDtypeStruct((M_PER, N), jnp.float32),
        in_specs=[
            pl.BlockSpec(memory_space=pltpu.VMEM),
            pl.BlockSpec(memory_space=pltpu.VMEM),
            pl.BlockSpec(memory_space=pltpu.SMEM),
            pl.BlockSpec(memory_space=pltpu.SMEM),
        ],
        out_specs=pl.BlockSpec(memory_space=pltpu.VMEM),
        scratch_shapes=[
            pltpu.VMEM((M_PER, K), jnp.int8),
            pltpu.SemaphoreType.DMA((N_DEV,)),
            pltpu.SemaphoreType.DMA((N_DEV,)),
        ],
    )(x, w_mat, scale_x, scale_w)


# device time: 20036 ns/iter; 1.4169x vs baseline; 1.4169x over previous
import jax
import jax.numpy as jnp
from jax import lax
from jax.experimental import pallas as pl
from jax.experimental.pallas import tpu as pltpu

N_DEV = 32
M_PER = 128
K = 4096
N = 2048


def kernel(x, w_mat, scale_x, scale_w):
    def body(x_ref, w_ref, sx_ref, sw_ref, out_ref, xl_ref,
             send_sems, recv_sems):
        me = lax.axis_index("i")

        partner = me + 1 - 2 * lax.rem(me, 2)
        barrier_sem = pltpu.get_barrier_semaphore()
        pl.semaphore_signal(
            barrier_sem, inc=1,
            device_id=(partner,), device_id_type=pl.DeviceIdType.MESH,
        )
        pl.semaphore_wait(barrier_sem, 1)

        rdmas = []
        for s in range(1, N_DEV):
            dst = lax.rem(me + s, N_DEV)
            rdma = pltpu.make_async_remote_copy(
                src_ref=x_ref.at[pl.ds(dst * M_PER, M_PER), :],
                dst_ref=xl_ref.at[:, pl.ds(me * M_PER, M_PER)],
                send_sem=send_sems.at[s],
                recv_sem=recv_sems.at[s],
                device_id=(dst,),
                device_id_type=pl.DeviceIdType.MESH,
            )
            rdma.start()
            rdmas.append(rdma)

        xl_ref[:, pl.ds(me * M_PER, M_PER)] = x_ref[pl.ds(me * M_PER, M_PER), :]

        for rdma in rdmas:
            rdma.wait_recv()
        for rdma in rdmas:
            rdma.wait_send()

        acc = lax.dot_general(
            xl_ref[:, :], w_ref[:, :],
            (((1,), (0,)), ((), ())),
            preferred_element_type=jnp.int32,
        )
        y = acc.astype(jnp.float32) * (sx_ref[0] * sw_ref[0])
        out_ref[:, :] = y * jax.nn.sigmoid(y)

    return pl.pallas_call(
        body,
        out_shape=jax.ShapeDtypeStruct((M_PER, N), jnp.float32),
        in_specs=[
            pl.BlockSpec(memory_space=pltpu.VMEM),
            pl.BlockSpec(memory_space=pltpu.VMEM),
            pl.BlockSpec(memory_space=pltpu.SMEM),
            pl.BlockSpec(memory_space=pltpu.SMEM),
        ],
        out_specs=pl.BlockSpec(memory_space=pltpu.VMEM),
        scratch_shapes=[
            pltpu.VMEM((M_PER, K), jnp.int8),
            pltpu.SemaphoreType.DMA((N_DEV,)),
            pltpu.SemaphoreType.DMA((N_DEV,)),
        ],
        compiler_params=pltpu.CompilerParams(collective_id=0),
    )(x, w_mat, scale_x, scale_w)
